# Initial kernel scaffold; baseline (speedup 1.0000x reference)
import jax
import jax.numpy as jnp
from jax.experimental import pallas as pl


def kernel(x, embeddings):
    raise NotImplementedError("write your pallas kernel here")


# speedup vs baseline: 1.2367x; 1.2367x over previous
import jax
import jax.numpy as jnp
from jax.experimental import pallas as pl
from jax.experimental.pallas import tpu as pltpu
from jax.experimental.pallas import tpu_sc as plsc


_BN = 512
_W = 4096


def _argmin_body(x_ref, e_ref, x2_ref, e2_ref, o_ref):
    xb = x_ref[...]
    eb = e_ref[...]
    xm = (2.0 * xb).astype(jnp.bfloat16)
    ebb = eb.astype(jnp.bfloat16)
    bn, d = xb.shape
    k = eb.shape[1]
    mm = jax.lax.dot_general(
        xm, ebb,
        dimension_numbers=(((1,), (0,)), ((), ())),
        preferred_element_type=jnp.float32,
    )
    x2 = x2_ref[0, 0, :]
    e2 = e2_ref[...]
    dis = (x2[:, None] + e2) - mm
    m = jnp.full((bn,), jnp.inf, jnp.float32)
    idx = jnp.zeros((bn,), jnp.int32)
    for s in range(0, k, _W):
        blk = dis[:, s:s + _W]
        vmin = jnp.min(blk, axis=1)
        iota = jax.lax.broadcasted_iota(jnp.int32, blk.shape, 1)
        li = jnp.min(jnp.where(blk == vmin[:, None], iota, _W), axis=1)
        upd = vmin < m
        m = jnp.where(upd, vmin.astype(jnp.bfloat16).astype(jnp.float32), m)
        idx = jnp.where(upd, s + li, idx)
    o_ref[0, 0, :] = idx


def _argmin_indices(x_flat, embeddings, x2, e2):
    n, d = x_flat.shape
    k = embeddings.shape[1]
    nb = n // _BN
    out = pl.pallas_call(
        _argmin_body,
        grid=(nb,),
        in_specs=[
            pl.BlockSpec((_BN, d), lambda i: (i, 0)),
            pl.BlockSpec((d, k), lambda i: (0, 0)),
            pl.BlockSpec((1, 1, _BN), lambda i: (i, 0, 0)),
            pl.BlockSpec((1, k), lambda i: (0, 0)),
        ],
        out_specs=pl.BlockSpec((1, 1, _BN), lambda i: (i, 0, 0)),
        out_shape=jax.ShapeDtypeStruct((nb, 1, _BN), jnp.int32),
    )(x_flat, embeddings, x2.reshape(nb, 1, _BN), e2.reshape(1, k))
    return out.reshape(n)


_GATHER_WINDOW = 128


def _sc_gather(table, indices):
    n = indices.shape[0]
    d = table.shape[1]
    idx2 = indices.reshape(1, n)
    mesh = plsc.VectorSubcoreMesh(core_axis_name="core",
                                  subcore_axis_name="subcore")

    @pl.kernel(out_type=jax.ShapeDtypeStruct((n, d), table.dtype), mesh=mesh)
    def gather_kernel(tab_hbm, i_hbm, o_hbm):
        def body(i_vmem, o_vmem):
            pltpu.sync_copy(tab_hbm.at[i_vmem.at[0]], o_vmem)

        pltpu.emit_pipeline(
            body,
            grid=(n // _GATHER_WINDOW,),
            in_specs=[pl.BlockSpec((1, _GATHER_WINDOW),
                                   index_map=lambda i: (0, i))],
            out_specs=[pl.BlockSpec((_GATHER_WINDOW, d),
                                    index_map=lambda i: (i, 0))],
            core_axis_name=("core", "subcore"),
            dimension_semantics=(pltpu.PARALLEL,),
        )(i_hbm, o_hbm)

    return gather_kernel(table, idx2)


def kernel(x, embeddings):
    b, hw, d = x.shape
    n = b * hw
    x_flat = x.reshape(n, d)
    x2 = jnp.square(x_flat).sum(-1)
    e2 = jnp.square(embeddings).sum(0)
    indices = _argmin_indices(x_flat, embeddings, x2, e2)
    table = jnp.pad(embeddings.T, ((0, 0), (0, 128 - d)))
    quantized = _sc_gather(table, indices)
    return quantized[:, :d].reshape(b, hw, d)
